# baseline (speedup 1.0000x reference)
import functools

import jax
import jax.numpy as jnp
from jax.experimental import pallas as pl
from jax.experimental.pallas import tpu as pltpu

INPUT_DIM = 1024
HIDDEN_DIM = 2048
OUTPUT_DIM = 1024
NUM_EXPERTS = 8
TOP_K = 2
BATCH = 4096

TILE = 256
NUM_TILES = (BATCH * TOP_K) // TILE + (NUM_EXPERTS - 1)
NUM_TILES = ((NUM_TILES + 7) // 8) * 8
SLOTS = NUM_TILES * TILE


def _ffn_body(meta_ref, xs_ref, w1_ref, w2_ref, b1_ref, b2_ref, ys_ref):
    i = pl.program_id(0)
    nreal = meta_ref[NUM_TILES]

    @pl.when(i < nreal)
    def _():
        h = jnp.dot(xs_ref[...], w1_ref[0],
                    preferred_element_type=jnp.float32)
        h = jnp.maximum(h + b1_ref[0, 0].astype(jnp.float32), 0.0)
        y = jnp.dot(h.astype(jnp.bfloat16), w2_ref[0],
                    preferred_element_type=jnp.float32)
        ys_ref[...] = y + b2_ref[0, 0].astype(jnp.float32)


def _grouped_ffn(xs, meta, W1b, b1, W2b, b2):
    grid_spec = pltpu.PrefetchScalarGridSpec(
        num_scalar_prefetch=1,
        grid=(NUM_TILES,),
        in_specs=[
            pl.BlockSpec((TILE, INPUT_DIM), lambda i, m: (i, 0)),
            pl.BlockSpec((1, INPUT_DIM, HIDDEN_DIM),
                         lambda i, m: (m[i], 0, 0)),
            pl.BlockSpec((1, HIDDEN_DIM, OUTPUT_DIM),
                         lambda i, m: (m[i], 0, 0)),
            pl.BlockSpec((1, 1, HIDDEN_DIM), lambda i, m: (m[i], 0, 0)),
            pl.BlockSpec((1, 1, OUTPUT_DIM), lambda i, m: (m[i], 0, 0)),
        ],
        out_specs=pl.BlockSpec((TILE, OUTPUT_DIM), lambda i, m: (i, 0)),
    )
    return pl.pallas_call(
        _ffn_body,
        grid_spec=grid_spec,
        out_shape=jax.ShapeDtypeStruct((SLOTS, OUTPUT_DIM), jnp.float32),
        compiler_params=pltpu.CompilerParams(
            dimension_semantics=("arbitrary",)),
    )(meta, xs, W1b, W2b, b1.reshape(NUM_EXPERTS, 1, HIDDEN_DIM),
      b2.reshape(NUM_EXPERTS, 1, OUTPUT_DIM))


def kernel(x, gate_w1, gate_b1, gate_w2, gate_b2, W1, b1, W2, b2):
    gh = jax.nn.relu(x @ gate_w1 + gate_b1)
    logits = gh @ gate_w2 + gate_b2
    gate_weights = jax.nn.softmax(logits, axis=1)
    top_k_weights, top_k_indices = jax.lax.top_k(gate_weights, TOP_K)
    top_k_weights = jax.nn.softmax(top_k_weights, axis=1)

    expert_fractions = gate_weights.mean(axis=0)
    cv_loss = jnp.sum((expert_fractions - 1.0 / NUM_EXPERTS) ** 2)

    e_flat = jnp.concatenate([top_k_indices[:, 0], top_k_indices[:, 1]])
    counts = jnp.bincount(e_flat, length=NUM_EXPERTS)
    ntiles = (counts + TILE - 1) // TILE
    tile_start = jnp.concatenate([jnp.zeros((1,), jnp.int32),
                                  jnp.cumsum(ntiles).astype(jnp.int32)])
    slot_start = tile_start * TILE
    csum_excl = jnp.concatenate([jnp.zeros((1,), jnp.int32),
                                 jnp.cumsum(counts).astype(jnp.int32)])[:-1]
    order = jnp.argsort(e_flat, stable=True)
    e_sorted = e_flat[order]
    within = jnp.arange(BATCH * TOP_K, dtype=jnp.int32) - csum_excl[e_sorted]
    slot_sorted = slot_start[e_sorted] + within
    slots = jnp.zeros((BATCH * TOP_K,), jnp.int32).at[order].set(slot_sorted)
    pos0, pos1 = slots[:BATCH], slots[BATCH:]

    t_iota = jnp.arange(NUM_TILES, dtype=jnp.int32)
    eot = (jnp.sum(t_iota[:, None] >= tile_start[None, 1:NUM_EXPERTS],
                   axis=1)).astype(jnp.int32)
    meta = jnp.concatenate([eot, tile_start[NUM_EXPERTS:]])

    xb = x.astype(jnp.bfloat16)
    tok = jnp.concatenate([jnp.arange(BATCH, dtype=jnp.int32)] * 2)
    xs = jnp.zeros((SLOTS, INPUT_DIM), jnp.bfloat16).at[slots].set(xb[tok])

    ys = _grouped_ffn(xs, meta, W1.astype(jnp.bfloat16), b1,
                      W2.astype(jnp.bfloat16), b2)

    output = (top_k_weights[:, 0:1] * ys[pos0]
              + top_k_weights[:, 1:2] * ys[pos1]).astype(x.dtype)
    return (output, gate_weights, cv_loss)


# speedup vs baseline: 1.0758x; 1.0758x over previous
import functools

import jax
import jax.numpy as jnp
from jax.experimental import pallas as pl
from jax.experimental.pallas import tpu as pltpu

INPUT_DIM = 1024
HIDDEN_DIM = 2048
OUTPUT_DIM = 1024
NUM_EXPERTS = 8
TOP_K = 2
BATCH = 4096

TILE = 512
NUM_TILES = (BATCH * TOP_K) // TILE + (NUM_EXPERTS - 1)
NUM_TILES = ((NUM_TILES + 7) // 8) * 8
SLOTS = NUM_TILES * TILE


def _ffn_body(meta_ref, xs_ref, w1_ref, w2_ref, b1_ref, b2_ref, ys_ref):
    i = pl.program_id(0)
    nreal = meta_ref[NUM_TILES]

    @pl.when(i < nreal)
    def _():
        h = jnp.dot(xs_ref[...], w1_ref[0],
                    preferred_element_type=jnp.float32)
        h = jnp.maximum(h + b1_ref[0, 0].astype(jnp.float32), 0.0)
        y = jnp.dot(h.astype(jnp.bfloat16), w2_ref[0],
                    preferred_element_type=jnp.float32)
        ys_ref[...] = y + b2_ref[0, 0].astype(jnp.float32)


def _grouped_ffn(xs, meta, W1b, b1, W2b, b2):
    grid_spec = pltpu.PrefetchScalarGridSpec(
        num_scalar_prefetch=1,
        grid=(NUM_TILES,),
        in_specs=[
            pl.BlockSpec((TILE, INPUT_DIM), lambda i, m: (i, 0)),
            pl.BlockSpec((1, INPUT_DIM, HIDDEN_DIM),
                         lambda i, m: (m[i], 0, 0)),
            pl.BlockSpec((1, HIDDEN_DIM, OUTPUT_DIM),
                         lambda i, m: (m[i], 0, 0)),
            pl.BlockSpec((1, 1, HIDDEN_DIM), lambda i, m: (m[i], 0, 0)),
            pl.BlockSpec((1, 1, OUTPUT_DIM), lambda i, m: (m[i], 0, 0)),
        ],
        out_specs=pl.BlockSpec((TILE, OUTPUT_DIM), lambda i, m: (i, 0)),
    )
    return pl.pallas_call(
        _ffn_body,
        grid_spec=grid_spec,
        out_shape=jax.ShapeDtypeStruct((SLOTS, OUTPUT_DIM), jnp.float32),
        compiler_params=pltpu.CompilerParams(
            dimension_semantics=("arbitrary",)),
    )(meta, xs, W1b, W2b, b1.reshape(NUM_EXPERTS, 1, HIDDEN_DIM),
      b2.reshape(NUM_EXPERTS, 1, OUTPUT_DIM))


def kernel(x, gate_w1, gate_b1, gate_w2, gate_b2, W1, b1, W2, b2):
    gh = jax.nn.relu(x @ gate_w1 + gate_b1)
    logits = gh @ gate_w2 + gate_b2
    gate_weights = jax.nn.softmax(logits, axis=1)
    top_k_weights, top_k_indices = jax.lax.top_k(gate_weights, TOP_K)
    top_k_weights = jax.nn.softmax(top_k_weights, axis=1)

    expert_fractions = gate_weights.mean(axis=0)
    cv_loss = jnp.sum((expert_fractions - 1.0 / NUM_EXPERTS) ** 2)

    e_flat = jnp.concatenate([top_k_indices[:, 0], top_k_indices[:, 1]])
    counts = jnp.bincount(e_flat, length=NUM_EXPERTS)
    ntiles = (counts + TILE - 1) // TILE
    tile_start = jnp.concatenate([jnp.zeros((1,), jnp.int32),
                                  jnp.cumsum(ntiles).astype(jnp.int32)])
    slot_start = tile_start * TILE
    csum_excl = jnp.concatenate([jnp.zeros((1,), jnp.int32),
                                 jnp.cumsum(counts).astype(jnp.int32)])[:-1]
    order = jnp.argsort(e_flat, stable=True)
    e_sorted = e_flat[order]
    within = jnp.arange(BATCH * TOP_K, dtype=jnp.int32) - csum_excl[e_sorted]
    slot_sorted = slot_start[e_sorted] + within
    slots = jnp.zeros((BATCH * TOP_K,), jnp.int32).at[order].set(slot_sorted)
    pos0, pos1 = slots[:BATCH], slots[BATCH:]

    t_iota = jnp.arange(NUM_TILES, dtype=jnp.int32)
    eot = (jnp.sum(t_iota[:, None] >= tile_start[None, 1:NUM_EXPERTS],
                   axis=1)).astype(jnp.int32)
    meta = jnp.concatenate([eot, tile_start[NUM_EXPERTS:]])

    xb = x.astype(jnp.bfloat16)
    tok = jnp.concatenate([jnp.arange(BATCH, dtype=jnp.int32)] * 2)
    xs = jnp.zeros((SLOTS, INPUT_DIM), jnp.bfloat16).at[slots].set(xb[tok])

    ys = _grouped_ffn(xs, meta, W1.astype(jnp.bfloat16), b1,
                      W2.astype(jnp.bfloat16), b2)

    output = (top_k_weights[:, 0:1] * ys[pos0]
              + top_k_weights[:, 1:2] * ys[pos1]).astype(x.dtype)
    return (output, gate_weights, cv_loss)
